# baseline (speedup 1.0000x reference)
import jax
import jax.numpy as jnp
import numpy as np
from jax.experimental import pallas as pl

_DROP = np.int32(2**30)
_VMEM_BUDGET = 48 * 1024 * 1024


def _mm(a, b, zero_diag=False):
    M, K = a.shape
    K2, N = b.shape
    assert K == K2

    def fits(bm, bn):
        est = 2 * (bm * K * a.dtype.itemsize + K * bn * b.dtype.itemsize)
        return est + bm * bn * 4 <= _VMEM_BUDGET

    cands = [c for c in (512, 256, 128) if M % c == 0]
    bn = next(c for c in (512, 256, 128) if N % c == 0)
    bm = next((c for c in cands if fits(c, bn)), cands[-1])
    if zero_diag:
        bn = bm if N % bm == 0 else bn
        bm = bn if M % bn == 0 else bm
        assert bm == bn

    def kern(a_ref, b_ref, o_ref):
        acc = jnp.dot(a_ref[...], b_ref[...],
                      preferred_element_type=jnp.float32)
        if zero_diag:
            i = pl.program_id(0)
            j = pl.program_id(1)
            rows = i * bm + jax.lax.broadcasted_iota(jnp.int32, (bm, bn), 0)
            cols = j * bn + jax.lax.broadcasted_iota(jnp.int32, (bm, bn), 1)
            acc = jnp.where(rows == cols, 0.0, acc)
        o_ref[...] = acc

    return pl.pallas_call(
        kern,
        grid=(M // bm, N // bn),
        in_specs=[
            pl.BlockSpec((bm, K), lambda i, j: (i, 0)),
            pl.BlockSpec((K, bn), lambda i, j: (0, j)),
        ],
        out_specs=pl.BlockSpec((bm, bn), lambda i, j: (i, j)),
        out_shape=jax.ShapeDtypeStruct((M, N), jnp.float32),
    )(a, b)


def _pad_rows(m, rows):
    return jnp.pad(m, ((0, rows - m.shape[0]), (0, 0)))


def _gcn_inner(A, AT, x, W, b, n):
    P = A.shape[0]
    deg = jnp.sum(AT, axis=1) + 2.0
    dis = deg ** -0.5
    z = dis[:, None] * _mm(x, W)
    out = dis[:, None] * (_mm(AT, z) + 2.0 * z) + b[None, :]
    mask = (jnp.arange(P) < n)[:, None]
    return jnp.where(mask, out, 0.0)


def _pool_stats(x, p, n, k):
    score = jnp.tanh((x[:n] @ p) / jnp.linalg.norm(p))
    vals, perm = jax.lax.top_k(score, k)
    return vals, perm


def _sel(perm, k_pad, n_pad):
    ar = jnp.arange(perm.shape[0], dtype=jnp.int32)
    return jnp.zeros((k_pad, n_pad), jnp.float32).at[ar, perm].set(1.0)


def _selT(perm, n_pad, k_pad):
    ar = jnp.arange(perm.shape[0], dtype=jnp.int32)
    return jnp.zeros((n_pad, k_pad), jnp.float32).at[perm, ar].set(1.0)


def _augment_pool(A, AT, S, use_bf16):
    if use_bf16:
        Sb = S.astype(jnp.bfloat16)
        R = _mm(Sb, A.astype(jnp.bfloat16)) + S
        CT = _mm(Sb, AT.astype(jnp.bfloat16)) + S
        Ap = _mm(R.astype(jnp.bfloat16), CT.astype(jnp.bfloat16).T,
                 zero_diag=True)
    else:
        R = _mm(S, A) + S
        CT = _mm(S, AT) + S
        Ap = _mm(R, CT.T, zero_diag=True)
    return Ap, Ap.T


def kernel(x, edge_index, params):
    N, C = x.shape
    ratio = 0.5

    def pad_to(v, m=512):
        return int(-(-v // m) * m)

    n0 = N
    k1 = int(np.ceil(ratio * n0))
    k2 = int(np.ceil(ratio * k1))
    k3 = int(np.ceil(ratio * k2))
    P0, P1, P2, P3 = pad_to(n0), pad_to(k1), pad_to(k2), pad_to(k3)

    src = edge_index[0]
    dst = edge_index[1]
    is_self = src == dst
    ones_e = jnp.ones_like(src, jnp.float32)

    A0T = jnp.zeros((P0, P0), jnp.float32).at[dst, src].add(ones_e)

    self_dst = jnp.where(is_self, dst, _DROP)
    self_cnt = jnp.zeros((n0,), jnp.float32).at[self_dst].add(ones_e)
    dst_ns = jnp.where(is_self, _DROP, dst)
    indeg_ns = jnp.zeros((n0,), jnp.float32).at[dst_ns].add(ones_e)
    diagval = jnp.where(self_cnt > 0, self_cnt, 2.0)
    deg0 = indeg_ns + diagval
    dis0 = deg0 ** -0.5
    dis0P = _pad_rows(dis0[:, None], P0)[:, 0]
    coef0 = _pad_rows((diagval - self_cnt)[:, None], P0)[:, 0]
    mask0 = (jnp.arange(P0) < n0)[:, None]

    def gcn0(xp, W, b):
        z = dis0P[:, None] * _mm(xp, W)
        out = dis0P[:, None] * (_mm(A0T, z) + coef0[:, None] * z) + b[None, :]
        return jnp.where(mask0, out, 0.0)

    xP = _pad_rows(x, P0)

    x1 = jax.nn.relu(gcn0(xP, params['Wd'][0], params['bd'][0]))

    vals1, perm1 = _pool_stats(x1, params['pw'][0], n0, k1)
    inv1 = jnp.full((n0,), _DROP, jnp.int32).at[perm1].set(
        jnp.arange(k1, dtype=jnp.int32))
    src_kept = jnp.where(is_self, _DROP, inv1[src])
    dst_kept = jnp.where(is_self, _DROP, inv1[dst])
    ar1 = jnp.arange(k1, dtype=jnp.int32)
    R = jnp.zeros((P1, P0), jnp.float32).at[src_kept, dst].add(ones_e)
    R = R.at[ar1, perm1].set(1.0)
    CT = jnp.zeros((P1, P0), jnp.float32).at[dst_kept, src].add(ones_e)
    CT = CT.at[ar1, perm1].set(1.0)
    A1 = _mm(R.astype(jnp.bfloat16), CT.astype(jnp.bfloat16).T,
             zero_diag=True)
    A1T = A1.T

    S1 = _sel(perm1, P1, P0)
    xp1 = _mm(S1, x1) * _pad_rows(vals1[:, None], P1)
    x2 = jax.nn.relu(_gcn_inner(A1, A1T, xp1,
                                params['Wd'][1], params['bd'][1], k1))

    vals2, perm2 = _pool_stats(x2, params['pw'][1], k1, k2)
    S2 = _sel(perm2, P2, P1)
    A2, A2T = _augment_pool(A1, A1T, S2, use_bf16=True)
    xp2 = _mm(S2, x2) * _pad_rows(vals2[:, None], P2)
    x3 = jax.nn.relu(_gcn_inner(A2, A2T, xp2,
                                params['Wd'][2], params['bd'][2], k2))

    vals3, perm3 = _pool_stats(x3, params['pw'][2], k2, k3)
    S3 = _sel(perm3, P3, P2)
    A3, A3T = _augment_pool(A2, A2T, S3, use_bf16=False)
    xp3 = _mm(S3, x3) * _pad_rows(vals3[:, None], P3)
    x4 = jax.nn.relu(_gcn_inner(A3, A3T, xp3,
                                params['Wd'][3], params['bd'][3], k3))

    u2 = x3 + _mm(_selT(perm3, P2, P3), x4)
    y2 = jax.nn.relu(_gcn_inner(A2, A2T, u2,
                                params['Wu'][0], params['bu'][0], k2))

    u1 = x2 + _mm(_selT(perm2, P1, P2), y2)
    y1 = jax.nn.relu(_gcn_inner(A1, A1T, u1,
                                params['Wu'][1], params['bu'][1], k1))

    u0 = x1 + _mm(_selT(perm1, P0, P1), y1)
    y0 = gcn0(u0, params['Wu'][2], params['bu'][2])

    return jax.nn.relu(y0[:N])


# speedup vs baseline: 1.2745x; 1.2745x over previous
import jax
import jax.numpy as jnp
import numpy as np
from jax.experimental import pallas as pl

_DROP = np.int32(2**30)
_VMEM_BUDGET = 48 * 1024 * 1024


def _mm(a, b, zero_diag=False):
    M, K = a.shape
    K2, N = b.shape
    assert K == K2

    def fits(bm, bn):
        est = 2 * (bm * K * a.dtype.itemsize + K * bn * b.dtype.itemsize)
        return est + bm * bn * 4 <= _VMEM_BUDGET

    cands = [c for c in (512, 256, 128) if M % c == 0]
    bn = next(c for c in (512, 256, 128) if N % c == 0)
    bm = next((c for c in cands if fits(c, bn)), cands[-1])
    if zero_diag:
        bn = bm if N % bm == 0 else bn
        bm = bn if M % bn == 0 else bm
        assert bm == bn

    def kern(a_ref, b_ref, o_ref):
        acc = jnp.dot(a_ref[...], b_ref[...],
                      preferred_element_type=jnp.float32)
        if zero_diag:
            i = pl.program_id(0)
            j = pl.program_id(1)
            rows = i * bm + jax.lax.broadcasted_iota(jnp.int32, (bm, bn), 0)
            cols = j * bn + jax.lax.broadcasted_iota(jnp.int32, (bm, bn), 1)
            acc = jnp.where(rows == cols, 0.0, acc)
        o_ref[...] = acc

    return pl.pallas_call(
        kern,
        grid=(M // bm, N // bn),
        in_specs=[
            pl.BlockSpec((bm, K), lambda i, j: (i, 0)),
            pl.BlockSpec((K, bn), lambda i, j: (0, j)),
        ],
        out_specs=pl.BlockSpec((bm, bn), lambda i, j: (i, j)),
        out_shape=jax.ShapeDtypeStruct((M, N), jnp.float32),
    )(a, b)


def _pad_rows(m, rows):
    return jnp.pad(m, ((0, rows - m.shape[0]), (0, 0)))


def _gcn_inner(A, AT, x, W, b, n):
    P = A.shape[0]
    deg = jnp.sum(AT, axis=1) + 2.0
    dis = deg ** -0.5
    z = dis[:, None] * _mm(x, W)
    out = dis[:, None] * (_mm(AT, z) + 2.0 * z) + b[None, :]
    mask = (jnp.arange(P) < n)[:, None]
    return jnp.where(mask, out, 0.0)


def _pool_stats(x, p, n, k):
    score = jnp.tanh((x[:n] @ p) / jnp.linalg.norm(p))
    vals, perm = jax.lax.top_k(score, k)
    return vals, perm


def _augment_pool(A, AT, perm, k_pad, dtype):
    P = A.shape[0]
    k = perm.shape[0]
    ar = jnp.arange(k, dtype=jnp.int32)
    R = jnp.zeros((k_pad, P), jnp.float32)
    R = R.at[:k, :].set(A[perm, :])
    R = R.at[ar, perm].set(1.0)
    CT = jnp.zeros((k_pad, P), jnp.float32)
    CT = CT.at[:k, :].set(AT[perm, :])
    CT = CT.at[ar, perm].set(1.0)
    Ap = _mm(R.astype(dtype), CT.astype(dtype).T, zero_diag=True)
    return Ap, Ap.T


def kernel(x, edge_index, params):
    N, C = x.shape
    ratio = 0.5

    def pad_to(v, m=512):
        return int(-(-v // m) * m)

    n0 = N
    k1 = int(np.ceil(ratio * n0))
    k2 = int(np.ceil(ratio * k1))
    k3 = int(np.ceil(ratio * k2))
    P0, P1, P2, P3 = pad_to(n0), pad_to(k1), pad_to(k2), pad_to(k3)

    src = edge_index[0]
    dst = edge_index[1]
    is_self = src == dst
    ones_e = jnp.ones_like(src, jnp.float32)

    A0T = jnp.zeros((P0, P0), jnp.float32).at[dst, src].add(ones_e)

    self_dst = jnp.where(is_self, dst, _DROP)
    self_cnt = jnp.zeros((n0,), jnp.float32).at[self_dst].add(ones_e)
    dst_ns = jnp.where(is_self, _DROP, dst)
    indeg_ns = jnp.zeros((n0,), jnp.float32).at[dst_ns].add(ones_e)
    diagval = jnp.where(self_cnt > 0, self_cnt, 2.0)
    deg0 = indeg_ns + diagval
    dis0 = deg0 ** -0.5
    dis0P = _pad_rows(dis0[:, None], P0)[:, 0]
    coef0 = _pad_rows((diagval - self_cnt)[:, None], P0)[:, 0]
    mask0 = (jnp.arange(P0) < n0)[:, None]

    def gcn0(xp, W, b):
        z = dis0P[:, None] * _mm(xp, W)
        out = dis0P[:, None] * (_mm(A0T, z) + coef0[:, None] * z) + b[None, :]
        return jnp.where(mask0, out, 0.0)

    xP = _pad_rows(x, P0)

    x1 = jax.nn.relu(gcn0(xP, params['Wd'][0], params['bd'][0]))

    vals1, perm1 = _pool_stats(x1, params['pw'][0], n0, k1)
    inv1 = jnp.full((n0,), _DROP, jnp.int32).at[perm1].set(
        jnp.arange(k1, dtype=jnp.int32))
    src_kept = jnp.where(is_self, _DROP, inv1[src])
    dst_kept = jnp.where(is_self, _DROP, inv1[dst])
    ar1 = jnp.arange(k1, dtype=jnp.int32)
    R = jnp.zeros((P1, P0), jnp.float32).at[src_kept, dst].add(ones_e)
    R = R.at[ar1, perm1].set(1.0)
    CT = jnp.zeros((P1, P0), jnp.float32).at[dst_kept, src].add(ones_e)
    CT = CT.at[ar1, perm1].set(1.0)
    A1 = _mm(R.astype(jnp.bfloat16), CT.astype(jnp.bfloat16).T,
             zero_diag=True)
    A1T = A1.T

    xp1 = _pad_rows(x1[perm1] * vals1[:, None], P1)
    x2 = jax.nn.relu(_gcn_inner(A1, A1T, xp1,
                                params['Wd'][1], params['bd'][1], k1))

    vals2, perm2 = _pool_stats(x2, params['pw'][1], k1, k2)
    A2, A2T = _augment_pool(A1, A1T, perm2, P2, jnp.bfloat16)
    xp2 = _pad_rows(x2[perm2] * vals2[:, None], P2)
    x3 = jax.nn.relu(_gcn_inner(A2, A2T, xp2,
                                params['Wd'][2], params['bd'][2], k2))

    vals3, perm3 = _pool_stats(x3, params['pw'][2], k2, k3)
    A3, A3T = _augment_pool(A2, A2T, perm3, P3, jnp.float32)
    xp3 = _pad_rows(x3[perm3] * vals3[:, None], P3)
    x4 = jax.nn.relu(_gcn_inner(A3, A3T, xp3,
                                params['Wd'][3], params['bd'][3], k3))

    u2 = x3 + jnp.zeros((P2, C), jnp.float32).at[perm3].set(x4[:k3])
    y2 = jax.nn.relu(_gcn_inner(A2, A2T, u2,
                                params['Wu'][0], params['bu'][0], k2))

    u1 = x2 + jnp.zeros((P1, C), jnp.float32).at[perm2].set(y2[:k2])
    y1 = jax.nn.relu(_gcn_inner(A1, A1T, u1,
                                params['Wu'][1], params['bu'][1], k1))

    u0 = x1 + jnp.zeros((P0, C), jnp.float32).at[perm1].set(y1[:k1])
    y0 = gcn0(u0, params['Wu'][2], params['bu'][2])

    return jax.nn.relu(y0[:N])
